# baseline (speedup 1.0000x reference)
import functools

import jax
import jax.numpy as jnp
from jax import lax
from jax.experimental import pallas as pl
from jax.experimental.pallas import tpu as pltpu
from jax.experimental.pallas import tpu_sc as plsc

_NUM_CORES = 2
_NUM_SUBCORES = 16
_NUM_WORKERS = _NUM_CORES * _NUM_SUBCORES
_CHUNK = 256
_NBUF = 3
_LANES = 128
_BLK = 16384


def _pad_body(t_ref, eye_ref, o_ref):
    del eye_ref
    xt = t_ref[...].T
    o_ref[...] = jnp.concatenate(
        [xt, jnp.zeros((xt.shape[0], _LANES - xt.shape[1]), jnp.float32)], axis=1
    )


@functools.partial(jax.jit, static_argnames=("vocab", "embed"))
def _pad_tc(table_t, eyepad, *, vocab, embed):
    return pl.pallas_call(
        _pad_body,
        grid=(pl.cdiv(vocab, _BLK),),
        in_specs=[
            pl.BlockSpec((embed, _BLK), lambda i: (0, i)),
            pl.BlockSpec((embed, _LANES), lambda i: (0, 0)),
        ],
        out_specs=pl.BlockSpec((_BLK, _LANES), lambda i: (i, 0)),
        out_shape=jax.ShapeDtypeStruct((vocab, _LANES), jnp.float32),
    )(table_t, eyepad)


@functools.partial(jax.jit, static_argnames=("seq", "embed"))
def _gather_sc(idx, table_pad, *, seq, embed):
    mesh = plsc.VectorSubcoreMesh(core_axis_name="c", subcore_axis_name="s")
    e_blocks = embed // 8

    @functools.partial(
        pl.kernel,
        out_type=jax.ShapeDtypeStruct(
            (seq, e_blocks, _NUM_WORKERS, 8, _LANES), jnp.float32
        ),
        mesh=mesh,
        compiler_params=pltpu.CompilerParams(
            use_tc_tiling_on_sc=False, needs_layout_passes=False
        ),
        scratch_types=[
            pltpu.VMEM((seq, _LANES), jnp.int32),
            pltpu.VMEM((_NBUF, _LANES, _LANES), jnp.float32),
            pltpu.VMEM((2, e_blocks, 8, _LANES), jnp.float32),
            pltpu.SemaphoreType.DMA((_NBUF,)),
            pltpu.SemaphoreType.DMA((2,)),
        ],
    )
    def k(idx_hbm, table_hbm, out_hbm, idx_v, rows_v, tr_v, gsem, osem):
        wid = lax.axis_index("s") * _NUM_CORES + lax.axis_index("c")
        pltpu.sync_copy(idx_hbm.at[wid], idx_v)
        lane_iota = lax.iota(jnp.int32, 16)

        def start_gather(l):
            b = l % _NBUF
            return pltpu.async_copy(
                table_hbm.at[idx_v.at[l]], rows_v.at[b], gsem.at[b]
            )

        def transpose(l):
            rb = l % _NBUF
            tb = l % 2

            def te(e, carry):
                col = jnp.full((16,), 0, jnp.int32) + e
                eb = e // 8
                e8 = e % 8
                for g in range(8):
                    v = plsc.load_gather(
                        rows_v.at[rb], [g * 16 + lane_iota, col]
                    )
                    tr_v[tb, eb, e8, pl.ds(g * 16, 16)] = v
                return carry

            lax.fori_loop(0, embed, te, 0)

        def start_out(l):
            tb = l % 2
            return [
                pltpu.async_copy(
                    tr_v.at[tb, eb], out_hbm.at[l, eb, wid], osem.at[tb]
                )
                for eb in range(e_blocks)
            ]

        gathers = [None] * seq
        outs = [None] * seq
        gathers[0] = start_gather(0)
        for l in range(seq):
            if l + 1 < seq:
                gathers[l + 1] = start_gather(l + 1)
            gathers[l].wait()
            if l >= 2:
                for c in outs[l - 2]:
                    c.wait()
            transpose(l)
            outs[l] = start_out(l)
        for l in (seq - 2, seq - 1):
            for c in outs[l]:
                c.wait()

    return k(idx, table_pad)


def kernel(token_ids, table):
    b, l = token_ids.shape
    vocab, embed = table.shape
    assert b % (_NUM_WORKERS * _LANES // 32) == 0 and embed % 8 == 0
    idx5 = (
        token_ids.astype(jnp.int32)
        .T.reshape(l, _NUM_WORKERS, _LANES)
        .transpose(1, 0, 2)
    )
    eyepad = jnp.eye(embed, _LANES, dtype=jnp.float32)
    table_pad = _pad_tc(table.T, eyepad, vocab=vocab, embed=embed)
    out5 = _gather_sc(idx5, table_pad, seq=l, embed=embed)
    return jnp.transpose(out5, (2, 4, 0, 1, 3)).reshape(b, l, embed)
